# Initial kernel scaffold; baseline (speedup 1.0000x reference)
import jax
import jax.numpy as jnp
from jax.experimental import pallas as pl


def kernel(xh0, edge_index, t, conditions, n_frag_switch, combined_mask, edge_attr, params):
    raise NotImplementedError("write your pallas kernel here")


# speedup vs baseline: 1.7171x; 1.7171x over previous
import functools
import jax
import jax.numpy as jnp
from jax import lax
from jax.experimental import pallas as pl
from jax.experimental.pallas import tpu as pltpu
from jax.experimental.pallas import tpu_sc as plsc

N = 10000
E = 320000
HC = 128
ENF = 16
NG = 16
PD = 16
HD = HC + PD
NC = 2
NS = 16
NW = NC * NS
EPW = E // NW
CH = 80
NPT = N // NS

BN = 2000
BE = 2000

f32 = jnp.float32


def _swish(x):
    return x * jax.nn.sigmoid(x)


def _dot(a, b):
    return jnp.dot(a, b, preferred_element_type=f32)


def _pre_body(feat_ref, pos_ref, t_ref, w1, b1, w2, b2, ew, e127, eb, out_ref):
    z = _swish(_dot(feat_ref[...], w1[...]) + b1[...])
    hp = _dot(z, w2[...]) + b2[...]
    h0 = _dot(hp, ew[...]) + t_ref[0, 0] * e127[...] + eb[...]
    out_ref[:, :HC] = h0
    out_ref[:, HC:] = pos_ref[...]


def _pre_call(feat, pos_pad, t2, w1, b1, w2p, b2p, ew, e127, eb):
    g = N // BN
    const = lambda shape: pl.BlockSpec(shape, lambda i: (0, 0))
    return pl.pallas_call(
        _pre_body,
        grid=(g,),
        in_specs=[
            pl.BlockSpec((BN, HC), lambda i: (i, 0)),
            pl.BlockSpec((BN, PD), lambda i: (i, 0)),
            pl.BlockSpec(memory_space=pltpu.SMEM),
            const((HC, 256)), const((1, 256)),
            const((256, HC)), const((1, HC)),
            const((HC, HC)), const((1, HC)), const((1, HC)),
        ],
        out_specs=pl.BlockSpec((BN, HD), lambda i: (i, 0)),
        out_shape=jax.ShapeDtypeStruct((N, HD), f32),
    )(feat, pos_pad, t2, w1, b1, w2p, b2p, ew, e127, eb)


def _gather_call(hcat, src_i, dst_i):
    mesh = plsc.VectorSubcoreMesh(core_axis_name="c", subcore_axis_name="s")

    @functools.partial(
        pl.kernel,
        out_type=(jax.ShapeDtypeStruct((E, HD), f32),
                  jax.ShapeDtypeStruct((E, HD), f32)),
        mesh=mesh,
        scratch_types=(pltpu.VMEM((CH,), jnp.int32),
                       pltpu.VMEM((CH, HD), f32),
                       pltpu.SemaphoreType.DMA),
        compiler_params=pltpu.CompilerParams(use_tc_tiling_on_sc=False),
    )
    def gather_k(hcat_ref, src_ref, dst_ref, osrc_ref, odst_ref,
                 idx_v, rows_v, sem):
        wid = lax.axis_index("s") * NC + lax.axis_index("c")
        base0 = wid * EPW

        def body(i, carry):
            b = base0 + i * CH
            pltpu.sync_copy(src_ref.at[pl.ds(b, CH)], idx_v)
            pltpu.async_copy(hcat_ref.at[idx_v], rows_v, sem).wait()
            pltpu.sync_copy(rows_v, osrc_ref.at[pl.ds(b, CH)])
            pltpu.sync_copy(dst_ref.at[pl.ds(b, CH)], idx_v)
            pltpu.async_copy(hcat_ref.at[idx_v], rows_v, sem).wait()
            pltpu.sync_copy(rows_v, odst_ref.at[pl.ds(b, CH)])
            return carry

        lax.fori_loop(0, EPW // CH, body, 0)

    return gather_k(hcat, src_i, dst_i)


def _edge_body(gs_ref, gd_ref, ea_ref, eew1, eeb1, eew2, eeb2,
               w1s, w1d, w1e, w1d2, b1, w2, b2, xw, xb, out_ref):
    hs = gs_ref[:, :HC]
    hd = gd_ref[:, :HC]
    rel = gs_ref[:, HC:] - gd_ref[:, HC:]
    d2 = jnp.sum(rel * rel, axis=1, keepdims=True)
    e = _dot(_swish(_dot(ea_ref[...], eew1[...]) + eeb1[...]), eew2[...]) + eeb2[...]
    pre = (_dot(hs, w1s[...]) + _dot(hd, w1d[...]) + _dot(e, w1e[...])
           + d2 * w1d2[...] + b1[...])
    m = _swish(_dot(_swish(pre), w2[...]) + b2[...])
    coef = _dot(m, xw[...]) + xb[...]
    out_ref[:, :HC] = m
    out_ref[:, HC:] = rel * (coef / (jnp.sqrt(d2) + 1.0))


def _edge_call(gsrc, gdst, edge_attr, eew1, eeb1, eew2, eeb2,
               w1s, w1d, w1e, w1d2, b1, w2, b2, xw, xb):
    g = E // BE
    const = lambda shape: pl.BlockSpec(shape, lambda i: (0, 0))
    return pl.pallas_call(
        _edge_body,
        grid=(g,),
        in_specs=[
            pl.BlockSpec((BE, HD), lambda i: (i, 0)),
            pl.BlockSpec((BE, HD), lambda i: (i, 0)),
            pl.BlockSpec((BE, ENF), lambda i: (i, 0)),
            const((ENF, 2 * ENF)), const((1, 2 * ENF)),
            const((2 * ENF, ENF)), const((1, ENF)),
            const((HC, HC)), const((HC, HC)), const((ENF, HC)),
            const((1, HC)), const((1, HC)),
            const((HC, HC)), const((1, HC)),
            const((HC, 1)), const((1, 1)),
        ],
        out_specs=pl.BlockSpec((BE, HD), lambda i: (i, 0)),
        out_shape=jax.ShapeDtypeStruct((E, HD), f32),
    )(gsrc, gdst, edge_attr, eew1, eeb1, eew2, eeb2,
      w1s, w1d, w1e, w1d2, b1, w2, b2, xw, xb)


def _scatter_call(mw, dst_i, zrows):
    mesh = plsc.VectorSubcoreMesh(core_axis_name="c", subcore_axis_name="s")

    @functools.partial(
        pl.kernel,
        out_type=jax.ShapeDtypeStruct((NC * N, HD), f32),
        mesh=mesh,
        scratch_types=(pltpu.VMEM((CH,), jnp.int32),
                       pltpu.VMEM((CH, HD), f32),
                       pltpu.VMEM_SHARED((N, HD), f32)),
        compiler_params=pltpu.CompilerParams(use_tc_tiling_on_sc=False),
    )
    def scatter_k(mw_ref, dst_ref, z_ref, out_ref, idx_v, rows_v, acc):
        c = lax.axis_index("c")
        s = lax.axis_index("s")
        wid = s * NC + c
        pltpu.sync_copy(z_ref, acc.at[pl.ds(s * NPT, NPT)])
        plsc.subcore_barrier()
        base0 = wid * EPW

        def body(i, carry):
            b = base0 + i * CH
            pltpu.sync_copy(dst_ref.at[pl.ds(b, CH)], idx_v)
            pltpu.sync_copy(mw_ref.at[pl.ds(b, CH)], rows_v)
            pltpu.sync_copy(rows_v, acc.at[idx_v], add=True)
            return carry

        lax.fori_loop(0, EPW // CH, body, 0)
        plsc.subcore_barrier()
        pltpu.sync_copy(acc.at[pl.ds(s * NPT, NPT)],
                        out_ref.at[pl.ds(c * N + s * NPT, NPT)])

    return scatter_k(mw, dst_i, zrows)


def _node_body(hc_ref, agg_ref, w1h, w1a, b1, w2, b2, out_ref):
    hc = hc_ref[...]
    h = hc[:, :HC]
    ag = agg_ref[0] + agg_ref[1]
    u = _swish(_dot(h, w1h[...]) + _dot(ag[:, :HC], w1a[...]) + b1[...])
    out_ref[:, :HC] = h + _dot(u, w2[...]) + b2[...]
    out_ref[:, HC:] = hc[:, HC:] + ag[:, HC:]


def _node_call(hcat, aggp, w1h, w1a, b1, w2, b2):
    g = N // BN
    const = lambda shape: pl.BlockSpec(shape, lambda i: (0, 0))
    return pl.pallas_call(
        _node_body,
        grid=(g,),
        in_specs=[
            pl.BlockSpec((BN, HD), lambda i: (i, 0)),
            pl.BlockSpec((NC, BN, HD), lambda i: (0, i, 0)),
            const((HC, HC)), const((HC, HC)), const((1, HC)),
            const((HC, HC)), const((1, HC)),
        ],
        out_specs=pl.BlockSpec((BN, HD), lambda i: (i, 0)),
        out_shape=jax.ShapeDtypeStruct((N, HD), f32),
    )(hcat, aggp, w1h, w1a, b1, w2, b2)


def _ro_body(hc_ref, mk_ref, w1, b1, w1g, b1g, w2, b2, w2g, b2g, w3, b3,
             conf_ref, s_acc, c_acc):
    i = pl.program_id(0)

    @pl.when(i == 0)
    def _():
        s_acc[...] = jnp.zeros_like(s_acc)
        c_acc[...] = jnp.zeros_like(c_acc)

    h = hc_ref[:, :HC]
    g1 = jax.nn.sigmoid(_dot(h, w1g[...]) + b1g[...])
    v = _swish((_dot(h, w1[...]) + b1[...]) * g1)
    g2 = jax.nn.sigmoid(_dot(v, w2g[...]) + b2g[...])
    v = _swish((_dot(v, w2[...]) + b2[...]) * g2)
    nout = _dot(v, w3[...]) + b3[...]
    oh = (mk_ref[...] == lax.broadcasted_iota(jnp.int32, (BN, NG), 1)).astype(f32)
    s_acc[...] += jnp.sum(oh * nout, axis=0, keepdims=True)
    c_acc[...] += jnp.sum(oh, axis=0, keepdims=True)
    conf_ref[...] = s_acc[...] / jnp.maximum(c_acc[...], 1.0)


def _ro_call(hcat, mask2, w1, b1, w1g, b1g, w2, b2, w2g, b2g, w3, b3):
    g = N // BN
    const = lambda shape: pl.BlockSpec(shape, lambda i: (0, 0))
    return pl.pallas_call(
        _ro_body,
        grid=(g,),
        in_specs=[
            pl.BlockSpec((BN, HD), lambda i: (i, 0)),
            pl.BlockSpec((BN, 1), lambda i: (i, 0)),
            const((HC, HC)), const((1, HC)),
            const((HC, HC)), const((1, HC)),
            const((HC, HC)), const((1, HC)),
            const((HC, HC)), const((1, HC)),
            const((HC, 1)), const((1, 1)),
        ],
        out_specs=pl.BlockSpec((1, NG), lambda i: (0, 0)),
        out_shape=jax.ShapeDtypeStruct((1, NG), f32),
        scratch_shapes=[pltpu.VMEM((1, NG), f32), pltpu.VMEM((1, NG), f32)],
    )(hcat, mask2, w1, b1, w1g, b1g, w2, b2, w2g, b2g, w3, b3)


def kernel(xh0, edge_index, t, conditions, n_frag_switch, combined_mask,
           edge_attr, params):
    p = params
    feat = xh0[:, 3:]
    pos_pad = jnp.pad(xh0[:, :3], ((0, 0), (0, PD - 3)))
    t2 = t.reshape(1, 1)
    src = edge_index[0]
    dst = edge_index[1]

    w2p = jnp.pad(p['enc_W2'], ((0, 0), (0, 1)))
    b2p = jnp.pad(p['enc_b2'], (0, 1)).reshape(1, HC)
    e127 = p['emb_W'][HC - 1:HC, :]

    hcat = _pre_call(feat, pos_pad, t2,
                     p['enc_W1'], p['enc_b1'].reshape(1, 256),
                     w2p, b2p,
                     p['emb_W'], e127, p['emb_b'].reshape(1, HC))

    zrows = jnp.zeros((NPT, HD), f32)
    for l in range(2):
        ew1 = p['l%d_eW1' % l]
        gsrc, gdst = _gather_call(hcat, src, dst)
        mw = _edge_call(gsrc, gdst, edge_attr,
                        p['ee_W1'], p['ee_b1'].reshape(1, 2 * ENF),
                        p['ee_W2'], p['ee_b2'].reshape(1, ENF),
                        ew1[:HC], ew1[HC:2 * HC], ew1[2 * HC + 1:],
                        ew1[2 * HC:2 * HC + 1],
                        p['l%d_eb1' % l].reshape(1, HC),
                        p['l%d_eW2' % l], p['l%d_eb2' % l].reshape(1, HC),
                        p['l%d_xW' % l], p['l%d_xb' % l].reshape(1, 1))
        aggf = _scatter_call(mw, dst, zrows)
        hw1 = p['l%d_hW1' % l]
        hcat = _node_call(hcat, aggf.reshape(NC, N, HD),
                          hw1[:HC], hw1[HC:],
                          p['l%d_hb1' % l].reshape(1, HC),
                          p['l%d_hW2' % l], p['l%d_hb2' % l].reshape(1, HC))

    conf = _ro_call(hcat, combined_mask.reshape(N, 1),
                    p['ro_W1'], p['ro_b1'].reshape(1, HC),
                    p['ro_W1g'], p['ro_b1g'].reshape(1, HC),
                    p['ro_W2'], p['ro_b2'].reshape(1, HC),
                    p['ro_W2g'], p['ro_b2g'].reshape(1, HC),
                    p['ro_W3'], p['ro_b3'].reshape(1, 1))
    return conf.reshape(NG, 1)


# speedup vs baseline: 2.1052x; 1.2260x over previous
import functools
import jax
import jax.numpy as jnp
from jax import lax
from jax.experimental import pallas as pl
from jax.experimental.pallas import tpu as pltpu
from jax.experimental.pallas import tpu_sc as plsc

N = 10000
E = 320000
HC = 128
ENF = 16
NG = 16
PD = 16
HD = HC + PD
NC = 2
NS = 16
NW = NC * NS
EPW = E // NW
CH = 80
NPT = N // NS

BN = 2000
BE = 2000

f32 = jnp.float32


def _swish(x):
    return x * jax.nn.sigmoid(x)


def _dot(a, b):
    return jnp.dot(a, b, preferred_element_type=f32)


def _pre_body(feat_ref, pos_ref, t_ref, w1, b1, w2, b2, ew, e127, eb, out_ref):
    z = _swish(_dot(feat_ref[...], w1[...]) + b1[...])
    hp = _dot(z, w2[...]) + b2[...]
    h0 = _dot(hp, ew[...]) + t_ref[0, 0] * e127[...] + eb[...]
    out_ref[:, :HC] = h0
    out_ref[:, HC:] = pos_ref[...]


def _pre_call(feat, pos_pad, t2, w1, b1, w2p, b2p, ew, e127, eb):
    g = N // BN
    const = lambda shape: pl.BlockSpec(shape, lambda i: (0, 0))
    return pl.pallas_call(
        _pre_body,
        grid=(g,),
        in_specs=[
            pl.BlockSpec((BN, HC), lambda i: (i, 0)),
            pl.BlockSpec((BN, PD), lambda i: (i, 0)),
            pl.BlockSpec(memory_space=pltpu.SMEM),
            const((HC, 256)), const((1, 256)),
            const((256, HC)), const((1, HC)),
            const((HC, HC)), const((1, HC)), const((1, HC)),
        ],
        out_specs=pl.BlockSpec((BN, HD), lambda i: (i, 0)),
        out_shape=jax.ShapeDtypeStruct((N, HD), f32),
    )(feat, pos_pad, t2, w1, b1, w2p, b2p, ew, e127, eb)


CHG = 200
NITG = EPW // CHG


def _gather_call(hcat, src_i, dst_i):
    mesh = plsc.VectorSubcoreMesh(core_axis_name="c", subcore_axis_name="s")

    @functools.partial(
        pl.kernel,
        out_type=(jax.ShapeDtypeStruct((E, HD), f32),
                  jax.ShapeDtypeStruct((E, HD), f32)),
        mesh=mesh,
        scratch_types=(pltpu.VMEM((EPW,), jnp.int32),
                       pltpu.VMEM((2, CHG, HD), f32),
                       pltpu.SemaphoreType.DMA((2,)),
                       pltpu.SemaphoreType.DMA((2,))),
        compiler_params=pltpu.CompilerParams(use_tc_tiling_on_sc=False),
    )
    def gather_k(hcat_ref, src_ref, dst_ref, osrc_ref, odst_ref,
                 idxb, buf, gsem, osem):
        wid = lax.axis_index("s") * NC + lax.axis_index("c")
        base0 = wid * EPW

        def phase(idx_hbm, out_hbm):
            pltpu.sync_copy(idx_hbm.at[pl.ds(base0, EPW)], idxb)

            def g_start(k, p):
                pltpu.async_copy(
                    hcat_ref.at[idxb.at[pl.ds(k * CHG, CHG)]],
                    buf.at[p], gsem.at[p])

            def g_wait(k, p):
                pltpu.make_async_copy(
                    hcat_ref.at[idxb.at[pl.ds(k * CHG, CHG)]],
                    buf.at[p], gsem.at[p]).wait()

            def o_start(k, p):
                pltpu.async_copy(
                    buf.at[p], out_hbm.at[pl.ds(base0 + k * CHG, CHG)],
                    osem.at[p])

            def o_wait(k):
                p = k % 2
                pltpu.make_async_copy(
                    buf.at[p], out_hbm.at[pl.ds(base0 + k * CHG, CHG)],
                    osem.at[p]).wait()

            g_start(0, 0)

            def body(k, carry):
                p = k % 2

                @pl.when(k + 1 < NITG)
                def _():
                    @pl.when(k >= 1)
                    def _():
                        o_wait(k - 1)
                    g_start(k + 1, 1 - p)

                g_wait(k, p)
                o_start(k, p)
                return carry

            lax.fori_loop(0, NITG, body, 0)
            o_wait(NITG - 2)
            o_wait(NITG - 1)

        phase(src_ref, osrc_ref)
        phase(dst_ref, odst_ref)

    return gather_k(hcat, src_i, dst_i)


def _edge_body(gs_ref, gd_ref, ea_ref, eew1, eeb1, eew2, eeb2,
               w1s, w1d, w1e, w1d2, b1, w2, b2, xw, xb, out_ref):
    hs = gs_ref[:, :HC]
    hd = gd_ref[:, :HC]
    rel = gs_ref[:, HC:] - gd_ref[:, HC:]
    d2 = jnp.sum(rel * rel, axis=1, keepdims=True)
    e = _dot(_swish(_dot(ea_ref[...], eew1[...]) + eeb1[...]), eew2[...]) + eeb2[...]
    pre = (_dot(hs, w1s[...]) + _dot(hd, w1d[...]) + _dot(e, w1e[...])
           + d2 * w1d2[...] + b1[...])
    m = _swish(_dot(_swish(pre), w2[...]) + b2[...])
    coef = _dot(m, xw[...]) + xb[...]
    out_ref[:, :HC] = m
    out_ref[:, HC:] = rel * (coef / (jnp.sqrt(d2) + 1.0))


def _edge_call(gsrc, gdst, edge_attr, eew1, eeb1, eew2, eeb2,
               w1s, w1d, w1e, w1d2, b1, w2, b2, xw, xb):
    g = E // BE
    const = lambda shape: pl.BlockSpec(shape, lambda i: (0, 0))
    return pl.pallas_call(
        _edge_body,
        grid=(g,),
        in_specs=[
            pl.BlockSpec((BE, HD), lambda i: (i, 0)),
            pl.BlockSpec((BE, HD), lambda i: (i, 0)),
            pl.BlockSpec((BE, ENF), lambda i: (i, 0)),
            const((ENF, 2 * ENF)), const((1, 2 * ENF)),
            const((2 * ENF, ENF)), const((1, ENF)),
            const((HC, HC)), const((HC, HC)), const((ENF, HC)),
            const((1, HC)), const((1, HC)),
            const((HC, HC)), const((1, HC)),
            const((HC, 1)), const((1, 1)),
        ],
        out_specs=pl.BlockSpec((BE, HD), lambda i: (i, 0)),
        out_shape=jax.ShapeDtypeStruct((E, HD), f32),
    )(gsrc, gdst, edge_attr, eew1, eeb1, eew2, eeb2,
      w1s, w1d, w1e, w1d2, b1, w2, b2, xw, xb)


def _scatter_call(mw, dst_i, zrows):
    mesh = plsc.VectorSubcoreMesh(core_axis_name="c", subcore_axis_name="s")

    @functools.partial(
        pl.kernel,
        out_type=jax.ShapeDtypeStruct((NC * N, HD), f32),
        mesh=mesh,
        scratch_types=(pltpu.VMEM((2, CH), jnp.int32),
                       pltpu.VMEM((2, CH, HD), f32),
                       pltpu.SemaphoreType.DMA((2,)),
                       pltpu.SemaphoreType.DMA((2,)),
                       pltpu.VMEM_SHARED((N, HD), f32)),
        compiler_params=pltpu.CompilerParams(use_tc_tiling_on_sc=False),
    )
    def scatter_k(mw_ref, dst_ref, z_ref, out_ref, ib, rb, lsem_i, lsem_r, acc):
        c = lax.axis_index("c")
        s = lax.axis_index("s")
        wid = s * NC + c
        pltpu.sync_copy(z_ref, acc.at[pl.ds(s * NPT, NPT)])
        plsc.subcore_barrier()
        base0 = wid * EPW
        nit = EPW // CH

        def l_start(k, p):
            b = base0 + k * CH
            pltpu.async_copy(dst_ref.at[pl.ds(b, CH)], ib.at[p], lsem_i.at[p])
            pltpu.async_copy(mw_ref.at[pl.ds(b, CH)], rb.at[p], lsem_r.at[p])

        def l_wait(k, p):
            b = base0 + k * CH
            pltpu.make_async_copy(dst_ref.at[pl.ds(b, CH)], ib.at[p],
                                  lsem_i.at[p]).wait()
            pltpu.make_async_copy(mw_ref.at[pl.ds(b, CH)], rb.at[p],
                                  lsem_r.at[p]).wait()

        l_start(0, 0)

        def body(k, carry):
            p = k % 2

            @pl.when(k + 1 < nit)
            def _():
                l_start(k + 1, 1 - p)

            l_wait(k, p)
            pltpu.sync_copy(rb.at[p], acc.at[ib.at[p]], add=True)
            return carry

        lax.fori_loop(0, nit, body, 0)
        plsc.subcore_barrier()
        pltpu.sync_copy(acc.at[pl.ds(s * NPT, NPT)],
                        out_ref.at[pl.ds(c * N + s * NPT, NPT)])

    return scatter_k(mw, dst_i, zrows)


def _node_body(hc_ref, agg_ref, w1h, w1a, b1, w2, b2, out_ref):
    hc = hc_ref[...]
    h = hc[:, :HC]
    ag = agg_ref[0] + agg_ref[1]
    u = _swish(_dot(h, w1h[...]) + _dot(ag[:, :HC], w1a[...]) + b1[...])
    out_ref[:, :HC] = h + _dot(u, w2[...]) + b2[...]
    out_ref[:, HC:] = hc[:, HC:] + ag[:, HC:]


def _node_call(hcat, aggp, w1h, w1a, b1, w2, b2):
    g = N // BN
    const = lambda shape: pl.BlockSpec(shape, lambda i: (0, 0))
    return pl.pallas_call(
        _node_body,
        grid=(g,),
        in_specs=[
            pl.BlockSpec((BN, HD), lambda i: (i, 0)),
            pl.BlockSpec((NC, BN, HD), lambda i: (0, i, 0)),
            const((HC, HC)), const((HC, HC)), const((1, HC)),
            const((HC, HC)), const((1, HC)),
        ],
        out_specs=pl.BlockSpec((BN, HD), lambda i: (i, 0)),
        out_shape=jax.ShapeDtypeStruct((N, HD), f32),
    )(hcat, aggp, w1h, w1a, b1, w2, b2)


def _ro_body(hc_ref, mk_ref, w1, b1, w1g, b1g, w2, b2, w2g, b2g, w3, b3,
             conf_ref, s_acc, c_acc):
    i = pl.program_id(0)

    @pl.when(i == 0)
    def _():
        s_acc[...] = jnp.zeros_like(s_acc)
        c_acc[...] = jnp.zeros_like(c_acc)

    h = hc_ref[:, :HC]
    g1 = jax.nn.sigmoid(_dot(h, w1g[...]) + b1g[...])
    v = _swish((_dot(h, w1[...]) + b1[...]) * g1)
    g2 = jax.nn.sigmoid(_dot(v, w2g[...]) + b2g[...])
    v = _swish((_dot(v, w2[...]) + b2[...]) * g2)
    nout = _dot(v, w3[...]) + b3[...]
    oh = (mk_ref[...] == lax.broadcasted_iota(jnp.int32, (BN, NG), 1)).astype(f32)
    s_acc[...] += jnp.sum(oh * nout, axis=0, keepdims=True)
    c_acc[...] += jnp.sum(oh, axis=0, keepdims=True)
    conf_ref[...] = s_acc[...] / jnp.maximum(c_acc[...], 1.0)


def _ro_call(hcat, mask2, w1, b1, w1g, b1g, w2, b2, w2g, b2g, w3, b3):
    g = N // BN
    const = lambda shape: pl.BlockSpec(shape, lambda i: (0, 0))
    return pl.pallas_call(
        _ro_body,
        grid=(g,),
        in_specs=[
            pl.BlockSpec((BN, HD), lambda i: (i, 0)),
            pl.BlockSpec((BN, 1), lambda i: (i, 0)),
            const((HC, HC)), const((1, HC)),
            const((HC, HC)), const((1, HC)),
            const((HC, HC)), const((1, HC)),
            const((HC, HC)), const((1, HC)),
            const((HC, 1)), const((1, 1)),
        ],
        out_specs=pl.BlockSpec((1, NG), lambda i: (0, 0)),
        out_shape=jax.ShapeDtypeStruct((1, NG), f32),
        scratch_shapes=[pltpu.VMEM((1, NG), f32), pltpu.VMEM((1, NG), f32)],
    )(hcat, mask2, w1, b1, w1g, b1g, w2, b2, w2g, b2g, w3, b3)


def kernel(xh0, edge_index, t, conditions, n_frag_switch, combined_mask,
           edge_attr, params):
    p = params
    feat = xh0[:, 3:]
    pos_pad = jnp.pad(xh0[:, :3], ((0, 0), (0, PD - 3)))
    t2 = t.reshape(1, 1)
    src = edge_index[0]
    dst = edge_index[1]

    w2p = jnp.pad(p['enc_W2'], ((0, 0), (0, 1)))
    b2p = jnp.pad(p['enc_b2'], (0, 1)).reshape(1, HC)
    e127 = p['emb_W'][HC - 1:HC, :]

    hcat = _pre_call(feat, pos_pad, t2,
                     p['enc_W1'], p['enc_b1'].reshape(1, 256),
                     w2p, b2p,
                     p['emb_W'], e127, p['emb_b'].reshape(1, HC))

    zrows = jnp.zeros((NPT, HD), f32)
    for l in range(2):
        ew1 = p['l%d_eW1' % l]
        gsrc, gdst = _gather_call(hcat, src, dst)
        mw = _edge_call(gsrc, gdst, edge_attr,
                        p['ee_W1'], p['ee_b1'].reshape(1, 2 * ENF),
                        p['ee_W2'], p['ee_b2'].reshape(1, ENF),
                        ew1[:HC], ew1[HC:2 * HC], ew1[2 * HC + 1:],
                        ew1[2 * HC:2 * HC + 1],
                        p['l%d_eb1' % l].reshape(1, HC),
                        p['l%d_eW2' % l], p['l%d_eb2' % l].reshape(1, HC),
                        p['l%d_xW' % l], p['l%d_xb' % l].reshape(1, 1))
        aggf = _scatter_call(mw, dst, zrows)
        hw1 = p['l%d_hW1' % l]
        hcat = _node_call(hcat, aggf.reshape(NC, N, HD),
                          hw1[:HC], hw1[HC:],
                          p['l%d_hb1' % l].reshape(1, HC),
                          p['l%d_hW2' % l], p['l%d_hb2' % l].reshape(1, HC))

    conf = _ro_call(hcat, combined_mask.reshape(N, 1),
                    p['ro_W1'], p['ro_b1'].reshape(1, HC),
                    p['ro_W1g'], p['ro_b1g'].reshape(1, HC),
                    p['ro_W2'], p['ro_b2'].reshape(1, HC),
                    p['ro_W2g'], p['ro_b2g'].reshape(1, HC),
                    p['ro_W3'], p['ro_b3'].reshape(1, 1))
    return conf.reshape(NG, 1)
